# baseline (speedup 1.0000x reference)
import functools

import jax
import jax.numpy as jnp
from jax.experimental import pallas as pl
from jax.experimental.pallas import tpu as pltpu


def _proj_kernel(x_ref, w_ref, b_ref, att1_ref, att2_ref, h_ref, a1_ref, a2_ref):
    h = (
        jnp.dot(x_ref[...], w_ref[...], preferred_element_type=jnp.float32)
        + b_ref[...]
    )
    h_ref[...] = h
    a1_ref[...] = jnp.dot(h, att1_ref[...], preferred_element_type=jnp.float32)
    a2_ref[...] = jnp.dot(h, att2_ref[...], preferred_element_type=jnp.float32)


def _project(x, w, b, att1, att2, row_block):
    n, c = x.shape
    grid = n // row_block
    h, a1, a2 = pl.pallas_call(
        _proj_kernel,
        grid=(grid,),
        in_specs=[
            pl.BlockSpec((row_block, c), lambda i: (i, 0)),
            pl.BlockSpec((c, c), lambda i: (0, 0)),
            pl.BlockSpec((1, c), lambda i: (0, 0)),
            pl.BlockSpec((c, 1), lambda i: (0, 0)),
            pl.BlockSpec((c, 1), lambda i: (0, 0)),
        ],
        out_specs=[
            pl.BlockSpec((row_block, c), lambda i: (i, 0)),
            pl.BlockSpec((row_block, 1), lambda i: (i, 0)),
            pl.BlockSpec((row_block, 1), lambda i: (i, 0)),
        ],
        out_shape=[
            jax.ShapeDtypeStruct((n, c), jnp.float32),
            jax.ShapeDtypeStruct((n, 1), jnp.float32),
            jax.ShapeDtypeStruct((n, 1), jnp.float32),
        ],
    )(x, w, b.reshape(1, c), att1.reshape(c, 1), att2.reshape(c, 1))
    return h, a1, a2


def _pack_lanes(a, n):
    npad = -n % 128
    flat = a.reshape(-1)
    if npad:
        flat = jnp.concatenate([flat, jnp.zeros((npad,), a.dtype)])
    return flat.reshape(-1, 128)


def _edge_kernel(
    n_edges, edge_block, src_ref, dst_ref, h_ref, asrc_ref, adst_ref, out_ref, den_ref
):
    i = pl.program_id(0)
    nb = pl.num_programs(0)

    @pl.when(i == 0)
    def _init():
        out_ref[...] = jnp.zeros_like(out_ref)
        den_ref[...] = jnp.zeros_like(den_ref)

    base = i * edge_block
    lane = jax.lax.broadcasted_iota(jnp.int32, (1, 128), 1)

    def body(k, carry):
        s = src_ref[k]
        d = dst_ref[k]
        arow = asrc_ref[pl.ds(s // 128, 1), :]
        a_s = jnp.sum(jnp.where(lane == s % 128, arow, 0.0), keepdims=True)
        brow = adst_ref[pl.ds(d // 128, 1), :]
        a_d = jnp.sum(jnp.where(lane == d % 128, brow, 0.0), keepdims=True)
        logit = a_s + a_d
        logit = jnp.where(logit >= 0, logit, 0.2 * logit)
        w = jnp.exp(logit)
        valid = (base + k) < n_edges
        w = jnp.where(valid, w, 0.0)
        row = h_ref[pl.ds(s, 1), :]
        out_ref[pl.ds(d, 1), :] = out_ref[pl.ds(d, 1), :] + row * w
        drow = den_ref[pl.ds(d // 128, 1), :]
        den_ref[pl.ds(d // 128, 1), :] = drow + jnp.where(lane == d % 128, w, 0.0)
        return carry

    jax.lax.fori_loop(0, edge_block, body, 0)

    @pl.when(i == nb - 1)
    def _epilogue():
        n_tiles = den_ref.shape[0]

        def norm(g, carry):
            denrow = den_ref[pl.ds(g, 1), :]
            den_col = jnp.transpose(jnp.broadcast_to(denrow, (128, 128)))
            blk = out_ref[pl.ds(g * 128, 128), :]
            out_ref[pl.ds(g * 128, 128), :] = jnp.maximum(
                blk / (den_col + 1e-16), 0.0
            )
            return carry

        jax.lax.fori_loop(0, n_tiles, norm, 0)


def _edge_pass(src, dst, h_src, a_src, a_dst, n_dst, edge_block):
    n_edges = src.shape[0]
    pad = (-n_edges) % edge_block
    if pad:
        src = jnp.concatenate([src, jnp.zeros((pad,), src.dtype)])
        dst = jnp.concatenate([dst, jnp.zeros((pad,), dst.dtype)])
    grid = src.shape[0] // edge_block
    n_src, c = h_src.shape
    asrc_p = _pack_lanes(a_src, n_src)
    adst_p = _pack_lanes(a_dst, n_dst)
    n_dst_pad = adst_p.shape[0] * 128
    out = pl.pallas_call(
        functools.partial(_edge_kernel, n_edges, edge_block),
        grid=(grid,),
        in_specs=[
            pl.BlockSpec((edge_block,), lambda i: (i,), memory_space=pltpu.SMEM),
            pl.BlockSpec((edge_block,), lambda i: (i,), memory_space=pltpu.SMEM),
            pl.BlockSpec((n_src, c), lambda i: (0, 0)),
            pl.BlockSpec(asrc_p.shape, lambda i: (0, 0)),
            pl.BlockSpec(adst_p.shape, lambda i: (0, 0)),
        ],
        out_specs=pl.BlockSpec((n_dst_pad, c), lambda i: (0, 0)),
        out_shape=jax.ShapeDtypeStruct((n_dst_pad, c), jnp.float32),
        scratch_shapes=[pltpu.VMEM(adst_p.shape, jnp.float32)],
    )(src, dst, h_src, asrc_p, adst_p)
    return out[:n_dst]


def kernel(
    x_ingredient,
    x_taste,
    edge_part_of,
    edge_contains,
    W_ing,
    b_ing,
    W_taste,
    b_taste,
    att_src_po,
    att_dst_po,
    att_src_co,
    att_dst_co,
    kW,
    kb,
    q,
):
    del kW, kb, q
    n_ing = x_ingredient.shape[0]
    n_taste = x_taste.shape[0]
    row_block = 1000
    edge_block = 4096

    h_ing, a_ing_po_src, a_ing_co_dst = _project(
        x_ingredient, W_ing, b_ing, att_src_po, att_dst_co, row_block
    )
    h_taste, a_taste_po_dst, a_taste_co_src = _project(
        x_taste, W_taste, b_taste, att_dst_po, att_src_co, row_block
    )

    out_taste = _edge_pass(
        edge_part_of[0],
        edge_part_of[1],
        h_ing,
        a_ing_po_src,
        a_taste_po_dst,
        n_taste,
        edge_block,
    )
    out_ing = _edge_pass(
        edge_contains[0],
        edge_contains[1],
        h_taste,
        a_taste_co_src,
        a_ing_co_dst,
        n_ing,
        edge_block,
    )
    return (out_ing, out_taste)


# speedup vs baseline: 11.9442x; 11.9442x over previous
import functools

import jax
import jax.numpy as jnp
from jax.experimental import pallas as pl
from jax.experimental.pallas import tpu as pltpu


def _proj_kernel(x_ref, w_ref, b_ref, att1_ref, att2_ref, h_ref, a1_ref, a2_ref):
    h = (
        jnp.dot(x_ref[...], w_ref[...], preferred_element_type=jnp.float32)
        + b_ref[...]
    )
    h_ref[...] = h
    a1_ref[...] = jnp.dot(h, att1_ref[...], preferred_element_type=jnp.float32)
    a2_ref[...] = jnp.dot(h, att2_ref[...], preferred_element_type=jnp.float32)


def _project(x, w, b, att1, att2, row_block):
    n, c = x.shape
    grid = n // row_block
    h, a1, a2 = pl.pallas_call(
        _proj_kernel,
        grid=(grid,),
        in_specs=[
            pl.BlockSpec((row_block, c), lambda i: (i, 0)),
            pl.BlockSpec((c, c), lambda i: (0, 0)),
            pl.BlockSpec((1, c), lambda i: (0, 0)),
            pl.BlockSpec((c, 1), lambda i: (0, 0)),
            pl.BlockSpec((c, 1), lambda i: (0, 0)),
        ],
        out_specs=[
            pl.BlockSpec((row_block, c), lambda i: (i, 0)),
            pl.BlockSpec((row_block, 1), lambda i: (i, 0)),
            pl.BlockSpec((row_block, 1), lambda i: (i, 0)),
        ],
        out_shape=[
            jax.ShapeDtypeStruct((n, c), jnp.float32),
            jax.ShapeDtypeStruct((n, 1), jnp.float32),
            jax.ShapeDtypeStruct((n, 1), jnp.float32),
        ],
    )(x, w, b.reshape(1, c), att1.reshape(c, 1), att2.reshape(c, 1))
    return h, a1, a2


def _pack_lanes(a, n):
    npad = -n % 128
    flat = a.reshape(-1)
    if npad:
        flat = jnp.concatenate([flat, jnp.zeros((npad,), a.dtype)])
    return flat.reshape(-1, 128)


def _edge_kernel(
    n_edges, edge_block, src_ref, dst_ref, h_ref, asrc_ref, adst_ref, out_ref, den_ref
):
    i = pl.program_id(0)
    nb = pl.num_programs(0)

    @pl.when(i == 0)
    def _init():
        out_ref[...] = jnp.zeros_like(out_ref)
        den_ref[...] = jnp.zeros_like(den_ref)

    base = i * edge_block
    lane = jax.lax.broadcasted_iota(jnp.int32, (1, 128), 1)

    def body(k, carry):
        s = src_ref[k]
        d = dst_ref[k]
        logit = asrc_ref[s] + adst_ref[d]
        logit = jnp.where(logit >= 0, logit, 0.2 * logit)
        w = jnp.exp(jnp.full((1, 1), logit, jnp.float32))
        valid = (base + k) < n_edges
        w = jnp.where(valid, w, 0.0)
        row = h_ref[pl.ds(s, 1), :]
        out_ref[pl.ds(d, 1), :] = out_ref[pl.ds(d, 1), :] + row * w
        drow = den_ref[pl.ds(d // 128, 1), :]
        den_ref[pl.ds(d // 128, 1), :] = drow + jnp.where(lane == d % 128, w, 0.0)
        return carry

    jax.lax.fori_loop(0, edge_block, body, 0, unroll=4)

    @pl.when(i == nb - 1)
    def _epilogue():
        n_tiles = den_ref.shape[0]

        def norm(g, carry):
            denrow = den_ref[pl.ds(g, 1), :]
            den_col = jnp.transpose(jnp.broadcast_to(denrow, (128, 128)))
            blk = out_ref[pl.ds(g * 128, 128), :]
            out_ref[pl.ds(g * 128, 128), :] = jnp.maximum(
                blk / (den_col + 1e-16), 0.0
            )
            return carry

        jax.lax.fori_loop(0, n_tiles, norm, 0)


def _edge_pass(src, dst, h_src, a_src, a_dst, n_dst, edge_block):
    n_edges = src.shape[0]
    pad = (-n_edges) % edge_block
    if pad:
        src = jnp.concatenate([src, jnp.zeros((pad,), src.dtype)])
        dst = jnp.concatenate([dst, jnp.zeros((pad,), dst.dtype)])
    grid = src.shape[0] // edge_block
    n_src, c = h_src.shape
    n_dst_tiles = (n_dst + 127) // 128
    n_dst_pad = n_dst_tiles * 128
    out = pl.pallas_call(
        functools.partial(_edge_kernel, n_edges, edge_block),
        grid=(grid,),
        in_specs=[
            pl.BlockSpec((edge_block,), lambda i: (i,), memory_space=pltpu.SMEM),
            pl.BlockSpec((edge_block,), lambda i: (i,), memory_space=pltpu.SMEM),
            pl.BlockSpec((n_src, c), lambda i: (0, 0)),
            pl.BlockSpec((n_src,), lambda i: (0,), memory_space=pltpu.SMEM),
            pl.BlockSpec((n_dst,), lambda i: (0,), memory_space=pltpu.SMEM),
        ],
        out_specs=pl.BlockSpec((n_dst_pad, c), lambda i: (0, 0)),
        out_shape=jax.ShapeDtypeStruct((n_dst_pad, c), jnp.float32),
        scratch_shapes=[pltpu.VMEM((n_dst_tiles, 128), jnp.float32)],
    )(src, dst, h_src, a_src.reshape(-1), a_dst.reshape(-1))
    return out[:n_dst]


def kernel(
    x_ingredient,
    x_taste,
    edge_part_of,
    edge_contains,
    W_ing,
    b_ing,
    W_taste,
    b_taste,
    att_src_po,
    att_dst_po,
    att_src_co,
    att_dst_co,
    kW,
    kb,
    q,
):
    del kW, kb, q
    n_ing = x_ingredient.shape[0]
    n_taste = x_taste.shape[0]
    row_block = 1000
    edge_block = 4096

    h_ing, a_ing_po_src, a_ing_co_dst = _project(
        x_ingredient, W_ing, b_ing, att_src_po, att_dst_co, row_block
    )
    h_taste, a_taste_po_dst, a_taste_co_src = _project(
        x_taste, W_taste, b_taste, att_dst_po, att_src_co, row_block
    )

    out_taste = _edge_pass(
        edge_part_of[0],
        edge_part_of[1],
        h_ing,
        a_ing_po_src,
        a_taste_po_dst,
        n_taste,
        edge_block,
    )
    out_ing = _edge_pass(
        edge_contains[0],
        edge_contains[1],
        h_taste,
        a_taste_co_src,
        a_ing_co_dst,
        n_ing,
        edge_block,
    )
    return (out_ing, out_taste)


# speedup vs baseline: 13.2170x; 1.1066x over previous
import functools

import jax
import jax.numpy as jnp
from jax.experimental import pallas as pl
from jax.experimental.pallas import tpu as pltpu


def _proj_kernel(x_ref, w_ref, b_ref, att1_ref, att2_ref, h_ref, a1_ref, a2_ref):
    h = (
        jnp.dot(x_ref[...], w_ref[...], preferred_element_type=jnp.float32)
        + b_ref[...]
    )
    h_ref[...] = h
    a1_ref[...] = jnp.dot(h, att1_ref[...], preferred_element_type=jnp.float32)
    a2_ref[...] = jnp.dot(h, att2_ref[...], preferred_element_type=jnp.float32)


def _project(x, w, b, att1, att2, row_block):
    n, c = x.shape
    grid = n // row_block
    h, a1, a2 = pl.pallas_call(
        _proj_kernel,
        grid=(grid,),
        in_specs=[
            pl.BlockSpec((row_block, c), lambda i: (i, 0)),
            pl.BlockSpec((c, c), lambda i: (0, 0)),
            pl.BlockSpec((1, c), lambda i: (0, 0)),
            pl.BlockSpec((c, 1), lambda i: (0, 0)),
            pl.BlockSpec((c, 1), lambda i: (0, 0)),
        ],
        out_specs=[
            pl.BlockSpec((row_block, c), lambda i: (i, 0)),
            pl.BlockSpec((row_block, 1), lambda i: (i, 0)),
            pl.BlockSpec((row_block, 1), lambda i: (i, 0)),
        ],
        out_shape=[
            jax.ShapeDtypeStruct((n, c), jnp.float32),
            jax.ShapeDtypeStruct((n, 1), jnp.float32),
            jax.ShapeDtypeStruct((n, 1), jnp.float32),
        ],
    )(x, w, b.reshape(1, c), att1.reshape(c, 1), att2.reshape(c, 1))
    return h, a1, a2


def _pack_lanes(a, n):
    npad = -n % 128
    flat = a.reshape(-1)
    if npad:
        flat = jnp.concatenate([flat, jnp.zeros((npad,), a.dtype)])
    return flat.reshape(-1, 128)


def _edge_kernel(
    n_edges, edge_block, src_ref, dst_ref, h_ref, asrc_ref, adst_ref, out_ref, den_ref
):
    i = pl.program_id(0)
    nb = pl.num_programs(0)

    @pl.when(i == 0)
    def _init():
        out_ref[...] = jnp.zeros_like(out_ref)
        den_ref[...] = jnp.zeros_like(den_ref)

    base = i * edge_block
    lane = jax.lax.broadcasted_iota(jnp.int32, (1, 128), 1)

    def body(k, carry):
        s = src_ref[k]
        d = dst_ref[k]
        logit = asrc_ref[s] + adst_ref[d]
        logit = jnp.where(logit >= 0, logit, 0.2 * logit)
        w = jnp.exp(jnp.full((1, 1), logit, jnp.float32))
        valid = (base + k) < n_edges
        w = jnp.where(valid, w, 0.0)
        row = h_ref[pl.ds(s, 1), :]
        out_ref[pl.ds(d, 1), :] = out_ref[pl.ds(d, 1), :] + row * w
        drow = den_ref[pl.ds(d // 128, 1), :]
        den_ref[pl.ds(d // 128, 1), :] = drow + jnp.where(lane == d % 128, w, 0.0)
        return carry

    jax.lax.fori_loop(0, edge_block, body, 0, unroll=8)

    @pl.when(i == nb - 1)
    def _epilogue():
        n_tiles = den_ref.shape[0]

        def norm(g, carry):
            denrow = den_ref[pl.ds(g, 1), :]
            den_col = jnp.transpose(jnp.broadcast_to(denrow, (128, 128)))
            blk = out_ref[pl.ds(g * 128, 128), :]
            out_ref[pl.ds(g * 128, 128), :] = jnp.maximum(
                blk / (den_col + 1e-16), 0.0
            )
            return carry

        jax.lax.fori_loop(0, n_tiles, norm, 0)


def _edge_pass(src, dst, h_src, a_src, a_dst, n_dst, edge_block):
    n_edges = src.shape[0]
    pad = (-n_edges) % edge_block
    if pad:
        src = jnp.concatenate([src, jnp.zeros((pad,), src.dtype)])
        dst = jnp.concatenate([dst, jnp.zeros((pad,), dst.dtype)])
    grid = src.shape[0] // edge_block
    n_src, c = h_src.shape
    n_dst_tiles = (n_dst + 127) // 128
    n_dst_pad = n_dst_tiles * 128
    out = pl.pallas_call(
        functools.partial(_edge_kernel, n_edges, edge_block),
        grid=(grid,),
        in_specs=[
            pl.BlockSpec((edge_block,), lambda i: (i,), memory_space=pltpu.SMEM),
            pl.BlockSpec((edge_block,), lambda i: (i,), memory_space=pltpu.SMEM),
            pl.BlockSpec((n_src, c), lambda i: (0, 0)),
            pl.BlockSpec((n_src,), lambda i: (0,), memory_space=pltpu.SMEM),
            pl.BlockSpec((n_dst,), lambda i: (0,), memory_space=pltpu.SMEM),
        ],
        out_specs=pl.BlockSpec((n_dst_pad, c), lambda i: (0, 0)),
        out_shape=jax.ShapeDtypeStruct((n_dst_pad, c), jnp.float32),
        scratch_shapes=[pltpu.VMEM((n_dst_tiles, 128), jnp.float32)],
    )(src, dst, h_src, a_src.reshape(-1), a_dst.reshape(-1))
    return out[:n_dst]


def kernel(
    x_ingredient,
    x_taste,
    edge_part_of,
    edge_contains,
    W_ing,
    b_ing,
    W_taste,
    b_taste,
    att_src_po,
    att_dst_po,
    att_src_co,
    att_dst_co,
    kW,
    kb,
    q,
):
    del kW, kb, q
    n_ing = x_ingredient.shape[0]
    n_taste = x_taste.shape[0]
    row_block = 1000
    edge_block = 4096

    h_ing, a_ing_po_src, a_ing_co_dst = _project(
        x_ingredient, W_ing, b_ing, att_src_po, att_dst_co, row_block
    )
    h_taste, a_taste_po_dst, a_taste_co_src = _project(
        x_taste, W_taste, b_taste, att_dst_po, att_src_co, row_block
    )

    out_taste = _edge_pass(
        edge_part_of[0],
        edge_part_of[1],
        h_ing,
        a_ing_po_src,
        a_taste_po_dst,
        n_taste,
        edge_block,
    )
    out_ing = _edge_pass(
        edge_contains[0],
        edge_contains[1],
        h_taste,
        a_taste_co_src,
        a_ing_co_dst,
        n_ing,
        edge_block,
    )
    return (out_ing, out_taste)
